# baseline (speedup 1.0000x reference)
import functools

import jax
import jax.numpy as jnp
from jax import lax
from jax.experimental import pallas as pl
from jax.experimental.pallas import tpu as pltpu
from jax.experimental.pallas import tpu_sc as plsc

LAT = 16
CH = 128


def _tc_relayout(*tables):
    V = tables[0].shape[0]
    W = 4096
    grid = (pl.cdiv(V, W),)
    n = len(tables)

    def body(*refs):
        for t in range(n):
            refs[n + t][...] = refs[t][...].T

    return pl.pallas_call(
        body,
        grid=grid,
        in_specs=[pl.BlockSpec((LAT, W), lambda i: (0, i))] * n,
        out_specs=[pl.BlockSpec((W, LAT), lambda i: (i, 0))] * n,
        out_shape=[jax.ShapeDtypeStruct((V, LAT), jnp.float32)] * n,
    )(*[jnp.transpose(t) for t in tables])


def _sc_gather(user_idx, item_idx, t_umlp, t_imlp, t_umf, t_imf):
    B = user_idx.shape[0]
    info = plsc.get_sparse_core_info()
    NC, NS = info.num_cores, info.num_subcores
    NW = NC * NS
    bpw = B // NW
    nch = bpw // CH
    mesh = plsc.VectorSubcoreMesh(core_axis_name="c", subcore_axis_name="s")

    @functools.partial(
        pl.kernel,
        mesh=mesh,
        compiler_params=pltpu.CompilerParams(use_tc_tiling_on_sc=False),
        out_type=(
            jax.ShapeDtypeStruct((B, LAT), jnp.float32),
            jax.ShapeDtypeStruct((B, LAT), jnp.float32),
            jax.ShapeDtypeStruct((B, LAT), jnp.float32),
        ),
        scratch_types=[
            pltpu.VMEM((nch, CH), jnp.int32),
            pltpu.VMEM((nch, CH), jnp.int32),
            pltpu.VMEM((bpw, LAT), jnp.float32),
            pltpu.VMEM((bpw, LAT), jnp.float32),
            pltpu.VMEM((bpw, LAT), jnp.float32),
            pltpu.VMEM((bpw, LAT), jnp.float32),
            pltpu.SemaphoreType.DMA,
        ],
    )
    def k(uidx_hbm, iidx_hbm, umlp_hbm, imlp_hbm, umf_hbm, imf_hbm,
          out_umlp, out_imlp, out_mf,
          uidx_v, iidx_v, ru, ri, rum, rim, sem):
        wid = lax.axis_index("s") * NC + lax.axis_index("c")
        base = wid * bpw
        for j in range(nch):
            pltpu.sync_copy(uidx_hbm.at[pl.ds(base + j * CH, CH)], uidx_v.at[j])
            pltpu.sync_copy(iidx_hbm.at[pl.ds(base + j * CH, CH)], iidx_v.at[j])
        copies = []
        for j in range(nch):
            sl = pl.ds(j * CH, CH)
            copies.append(pltpu.async_copy(umlp_hbm.at[uidx_v.at[j]], ru.at[sl], sem))
            copies.append(pltpu.async_copy(imlp_hbm.at[iidx_v.at[j]], ri.at[sl], sem))
            copies.append(pltpu.async_copy(umf_hbm.at[uidx_v.at[j]], rum.at[sl], sem))
            copies.append(pltpu.async_copy(imf_hbm.at[iidx_v.at[j]], rim.at[sl], sem))
        for c in copies:
            c.wait()

        def body(r4, carry):
            for t in range(4):
                r = r4 * 4 + t
                rum[r, :] = rum[r, :] * rim[r, :]
            return carry

        lax.fori_loop(0, bpw // 4, body, 0)
        pltpu.sync_copy(ru, out_umlp.at[pl.ds(base, bpw)])
        pltpu.sync_copy(ri, out_imlp.at[pl.ds(base, bpw)])
        pltpu.sync_copy(rum, out_mf.at[pl.ds(base, bpw)])

    return k(user_idx, item_idx, t_umlp, t_imlp, t_umf, t_imf)


def _tc_mlp(u_mlp, i_mlp, mf, W1, b1, W2, b2, W_out, b_out):
    B = u_mlp.shape[0]
    BLK = 2048
    HID = LAT // 2
    W1a = W1[:LAT]
    W1b = W1[LAT:]
    b1r = b1.reshape(1, LAT)
    b2r = b2.reshape(1, HID)
    wh2 = W_out[:HID, 0].reshape(1, HID)
    wmf = W_out[HID:, 0].reshape(1, LAT)
    bor = b_out.reshape(1, 1)

    def body(u_ref, i_ref, mf_ref, w1a, w1b, b1_, w2, b2_, wh2_, wmf_, bo, out_ref):
        h1 = jnp.maximum(
            jnp.dot(u_ref[...], w1a[...], preferred_element_type=jnp.float32)
            + jnp.dot(i_ref[...], w1b[...], preferred_element_type=jnp.float32)
            + b1_[...], 0.0)
        h2 = jnp.maximum(
            jnp.dot(h1, w2[...], preferred_element_type=jnp.float32) + b2_[...], 0.0)
        logit = (jnp.sum(h2 * wh2_[...], axis=1, keepdims=True)
                 + jnp.sum(mf_ref[...] * wmf_[...], axis=1, keepdims=True)
                 + bo[...])
        out_ref[...] = jax.nn.sigmoid(logit)

    row = lambda i: (i, 0)
    rep = lambda i: (0, 0)
    return pl.pallas_call(
        body,
        grid=(B // BLK,),
        in_specs=[
            pl.BlockSpec((BLK, LAT), row),
            pl.BlockSpec((BLK, LAT), row),
            pl.BlockSpec((BLK, LAT), row),
            pl.BlockSpec((LAT, LAT), rep),
            pl.BlockSpec((LAT, LAT), rep),
            pl.BlockSpec((1, LAT), rep),
            pl.BlockSpec((LAT, HID), rep),
            pl.BlockSpec((1, HID), rep),
            pl.BlockSpec((1, HID), rep),
            pl.BlockSpec((1, LAT), rep),
            pl.BlockSpec((1, 1), rep),
        ],
        out_specs=pl.BlockSpec((BLK, 1), row),
        out_shape=jax.ShapeDtypeStruct((B, 1), jnp.float32),
    )(u_mlp, i_mlp, mf, W1a, W1b, b1r, W2, b2r, wh2, wmf, bor)


def kernel(user_indices, item_indices, emb_user_mlp, emb_item_mlp,
           emb_user_mf, emb_item_mf, W1, b1, W2, b2, W_out, b_out):
    t_umlp, t_imlp, t_umf, t_imf = _tc_relayout(
        emb_user_mlp, emb_item_mlp, emb_user_mf, emb_item_mf)
    u_mlp, i_mlp, mf = _sc_gather(user_indices, item_indices,
                                  t_umlp, t_imlp, t_umf, t_imf)
    return _tc_mlp(u_mlp, i_mlp, mf, W1, b1, W2, b2, W_out, b_out)


# speedup vs baseline: 3.8050x; 3.8050x over previous
import functools

import jax
import jax.numpy as jnp
from jax import lax
from jax.experimental import pallas as pl
from jax.experimental.pallas import tpu as pltpu
from jax.experimental.pallas import tpu_sc as plsc

LAT = 16
CH = 128


def _tc_detile(*tables):
    V = tables[0].shape[0]
    W = 16384
    grid = (pl.cdiv(V, W),)
    n = len(tables)
    views = [jnp.transpose(t).reshape(LAT, 1, V) for t in tables]

    def body(*refs):
        for s in range(n * LAT):
            refs[n * LAT + s][...] = refs[s][0, 0, :]

    in_specs = []
    for _ in range(n):
        for f in range(LAT):
            in_specs.append(
                pl.BlockSpec((1, 1, W), lambda i, f=f: (f, 0, i)))
    return pl.pallas_call(
        body,
        grid=grid,
        in_specs=in_specs,
        out_specs=[pl.BlockSpec((W,), lambda i: (i,))] * (n * LAT),
        out_shape=[jax.ShapeDtypeStruct((V,), jnp.float32)] * (n * LAT),
    )(*[v for v in views for _ in range(LAT)])


def _sc_gather_t(user_idx, item_idx, slabs):
    B = user_idx.shape[0]
    info = plsc.get_sparse_core_info()
    NC, NS = info.num_cores, info.num_subcores
    NW = NC * NS
    bpw = B // NW
    nch = bpw // CH
    mesh = plsc.VectorSubcoreMesh(core_axis_name="c", subcore_axis_name="s")

    @functools.partial(
        pl.kernel,
        mesh=mesh,
        out_type=(
            jax.ShapeDtypeStruct((LAT, B // CH, CH), jnp.float32),
            jax.ShapeDtypeStruct((LAT, B // CH, CH), jnp.float32),
            jax.ShapeDtypeStruct((LAT, B // CH, CH), jnp.float32),
        ),
        scratch_types=[
            pltpu.VMEM((nch, CH), jnp.int32),
            pltpu.VMEM((nch, CH), jnp.int32),
            pltpu.VMEM((LAT, nch, CH), jnp.float32),
            pltpu.VMEM((LAT, nch, CH), jnp.float32),
            pltpu.VMEM((LAT, nch, CH), jnp.float32),
            pltpu.VMEM((LAT, nch, CH), jnp.float32),
            pltpu.SemaphoreType.DMA,
        ],
    )
    def k(uidx_hbm, iidx_hbm, *rest):
        slab_refs = rest[:4 * LAT]
        out_umlp, out_imlp, out_mf = rest[4 * LAT:4 * LAT + 3]
        uidx_v, iidx_v, ru, ri, rum, rim, sem = rest[4 * LAT + 3:]
        wid = lax.axis_index("s") * NC + lax.axis_index("c")
        base = wid * bpw
        for c in range(nch):
            pltpu.sync_copy(uidx_hbm.at[pl.ds(base + c * CH, CH)],
                            uidx_v.at[c])
            pltpu.sync_copy(iidx_hbm.at[pl.ds(base + c * CH, CH)],
                            iidx_v.at[c])
        pending = []
        for t, (buf, idxref) in enumerate(((ru, uidx_v), (ri, iidx_v),
                                           (rum, uidx_v), (rim, iidx_v))):
            issued = []
            for f in range(LAT):
                slab = slab_refs[t * LAT + f]
                for c in range(nch):
                    issued.append(pltpu.async_copy(
                        slab.at[idxref.at[c]], buf.at[f, c], sem))
            for cp in pending:
                cp.wait()
            pending = issued
        for cp in pending:
            cp.wait()

        def prod(c, carry):
            for f in range(LAT):
                for q in range(CH // LAT):
                    sl = pl.ds(q * LAT, LAT)
                    rum[f, c, sl] = rum[f, c, sl] * rim[f, c, sl]
            return carry

        lax.fori_loop(0, nch, prod, 0)
        cols = pl.ds(wid * nch, nch)
        pltpu.sync_copy(ru, out_umlp.at[:, cols])
        pltpu.sync_copy(ri, out_imlp.at[:, cols])
        pltpu.sync_copy(rum, out_mf.at[:, cols])

    outs = k(user_idx, item_idx, *slabs)
    return [o.reshape(LAT, B) for o in outs]


def _tc_mlp_t(uT, iT, mfT, W1, b1, W2, b2, W_out, b_out):
    B = uT.shape[1]
    BLK = 2048
    HID = LAT // 2
    W1aT = W1[:LAT].T
    W1bT = W1[LAT:].T
    W2T = W2.T
    b1c = b1.reshape(LAT, 1)
    b2c = b2.reshape(HID, 1)
    wh2 = W_out[:HID, 0].reshape(1, HID)
    wmf = W_out[HID:, 0].reshape(1, LAT)
    bor = b_out.reshape(1, 1)

    def body(u_ref, i_ref, mf_ref, w1a, w1b, b1_, w2, b2_, wh2_, wmf_, bo,
             out_ref):
        h1 = jnp.maximum(
            jnp.dot(w1a[...], u_ref[...], preferred_element_type=jnp.float32)
            + jnp.dot(w1b[...], i_ref[...], preferred_element_type=jnp.float32)
            + b1_[...], 0.0)
        h2 = jnp.maximum(
            jnp.dot(w2[...], h1, preferred_element_type=jnp.float32)
            + b2_[...], 0.0)
        logit = (jnp.dot(wh2_[...], h2, preferred_element_type=jnp.float32)
                 + jnp.dot(wmf_[...], mf_ref[...],
                           preferred_element_type=jnp.float32)
                 + bo[...])
        out_ref[...] = jax.nn.sigmoid(logit)

    col = lambda i: (0, i)
    rep = lambda i: (0, 0)
    return pl.pallas_call(
        body,
        grid=(B // BLK,),
        in_specs=[
            pl.BlockSpec((LAT, BLK), col),
            pl.BlockSpec((LAT, BLK), col),
            pl.BlockSpec((LAT, BLK), col),
            pl.BlockSpec((LAT, LAT), rep),
            pl.BlockSpec((LAT, LAT), rep),
            pl.BlockSpec((LAT, 1), rep),
            pl.BlockSpec((HID, LAT), rep),
            pl.BlockSpec((HID, 1), rep),
            pl.BlockSpec((1, HID), rep),
            pl.BlockSpec((1, LAT), rep),
            pl.BlockSpec((1, 1), rep),
        ],
        out_specs=pl.BlockSpec((1, BLK), col),
        out_shape=jax.ShapeDtypeStruct((1, B), jnp.float32),
    )(uT, iT, mfT, W1aT, W1bT, b1c, W2T, b2c, wh2, wmf, bor)


def kernel(user_indices, item_indices, emb_user_mlp, emb_item_mlp,
           emb_user_mf, emb_item_mf, W1, b1, W2, b2, W_out, b_out):
    slabs = _tc_detile(emb_user_mlp, emb_item_mlp, emb_user_mf, emb_item_mf)
    uT, iT, mfT = _sc_gather_t(user_indices, item_indices, slabs)
    out = _tc_mlp_t(uT, iT, mfT, W1, b1, W2, b2, W_out, b_out)
    return out.reshape(user_indices.shape[0], 1)
